# baseline (speedup 1.0000x reference)
import jax
import jax.numpy as jnp
from jax import lax
from jax.experimental import pallas as pl
from jax.experimental.pallas import tpu as pltpu
from jax.experimental.pallas import tpu_sc as plsc

_V = 100000
_K = 16
_L = 16
_THRESH = 10

_TCOLS = _V // 128
_TAIL = _V - _TCOLS * 128
_VPAD = (_TCOLS + 1) * 128


def _bm_select(xs, ones, neg1):
    cand = xs[0]
    cnt = ones
    for k in range(1, _K):
        xk = xs[k]
        eq = xk == cand
        dead = cnt == 0
        delta = jnp.where(eq, ones, neg1)
        cnt2 = cnt + delta
        cand = jnp.where(dead, xk, cand)
        cnt = jnp.where(dead, ones, cnt2)
    eqs = [jnp.where(xs[k] == cand, ones, neg1) for k in range(_K)]
    while len(eqs) > 1:
        eqs = [a + b for a, b in zip(eqs[::2], eqs[1::2])]
    thresh = ones * (2 * _THRESH - _K)
    return jnp.where(eqs[0] >= thresh, cand, neg1)


def _make_body(nc, nw):
    q, r = divmod(_TCOLS, nw)
    big_w, small_w = (q + 1) * 128, q * 128
    half1 = (q // 2) * 128
    big_w2, small_w2 = big_w - half1, small_w - half1
    p1 = half1 // 32
    p2 = big_w // 32 - p1

    def body(in_hbm, tail_hbm, out_hbm, buf, out_v, tail_buf, tail_out,
             sem1, sem2):
        c = lax.axis_index("c")
        s = lax.axis_index("s")
        wid = s * nc + c
        is_big = wid < r
        col_base = jnp.where(is_big, wid * big_w,
                             r * big_w + (wid - r) * small_w)

        cp1 = pltpu.make_async_copy(in_hbm.at[:, pl.ds(col_base, half1)],
                                    buf.at[:, pl.ds(0, half1)], sem1)
        cp1.start()
        cp2b = pltpu.make_async_copy(
            in_hbm.at[:, pl.ds(col_base + half1, big_w2)],
            buf.at[:, pl.ds(half1, big_w2)], sem2)
        cp2s = pltpu.make_async_copy(
            in_hbm.at[:, pl.ds(col_base + half1, small_w2)],
            buf.at[:, pl.ds(half1, small_w2)], sem2)

        @pl.when(is_big)
        def _():
            cp2b.start()

        @pl.when(jnp.logical_not(is_big))
        def _():
            cp2s.start()

        ones16 = jnp.full((2 * _L,), 1, jnp.int16)
        neg16 = jnp.full((2 * _L,), -1, jnp.int16)

        def do_pair(p):
            xs = []
            for k in range(_K):
                a = buf[k, pl.ds(p * 32, _L)]
                b = buf[k, pl.ds(p * 32 + _L, _L)]
                xs.append(plsc.pack(a, b, format=plsc.PackFormat.INTERLEAVED))
            res = _bm_select(xs, ones16, neg16)
            ra, rb = plsc.unpack(res, format=plsc.PackFormat.INTERLEAVED)
            ra = (ra << 16) >> 16
            rb = (rb << 16) >> 16
            out_v[0, pl.ds(p * 32, _L)] = ra
            out_v[0, pl.ds(p * 32 + _L, _L)] = rb

        cp1.wait()
        plsc.parallel_loop(0, p1, unroll=2)(do_pair)

        @pl.when(is_big)
        def _():
            cp2b.wait()

        @pl.when(jnp.logical_not(is_big))
        def _():
            cp2s.wait()

        plsc.parallel_loop(p1, p1 + p2, unroll=2)(do_pair)

        @pl.when(is_big)
        def _():
            pltpu.sync_copy(out_v, out_hbm.at[:, pl.ds(col_base, big_w)])

        @pl.when(jnp.logical_not(is_big))
        def _():
            pltpu.sync_copy(out_v.at[:, pl.ds(0, small_w)],
                            out_hbm.at[:, pl.ds(col_base, small_w)])

        @pl.when(wid == nw - 1)
        def _():
            pltpu.sync_copy(tail_hbm, tail_buf)
            ones32 = jnp.full((_L,), 1, jnp.int32)
            neg32 = jnp.full((_L,), -1, jnp.int32)
            for g in range(_TAIL // _L):
                xs = [tail_buf[k, pl.ds(g * _L, _L)] for k in range(_K)]
                tail_out[0, pl.ds(g * _L, _L)] = _bm_select(xs, ones32, neg32)
            pltpu.sync_copy(tail_out, out_hbm.at[:, pl.ds(_TCOLS * 128, 128)])

    return body


def kernel(inputs):
    info = plsc.get_sparse_core_info()
    nc, ns = info.num_cores, info.num_subcores
    nw = nc * ns
    q, r = divmod(_TCOLS, nw)
    big_w = (q + 1) * 128

    body = _make_body(nc, nw)
    mesh = plsc.VectorSubcoreMesh(core_axis_name="c", subcore_axis_name="s")
    xt = inputs.T
    tail = xt[:, _TCOLS * 128:]
    out = pl.kernel(
        body,
        out_type=jax.ShapeDtypeStruct((1, _VPAD), jnp.int32),
        mesh=mesh,
        scratch_types=[
            pltpu.VMEM((_K, big_w), jnp.int32),
            pltpu.VMEM((1, big_w), jnp.int32),
            pltpu.VMEM((_K, _TAIL), jnp.int32),
            pltpu.VMEM((1, 128), jnp.int32),
            pltpu.SemaphoreType.DMA,
            pltpu.SemaphoreType.DMA,
        ],
        compiler_params=pltpu.CompilerParams(
            use_tc_tiling_on_sc=True,
            needs_layout_passes=False,
        ),
    )(xt, tail)
    return out[0, :_V].reshape(_V, 1)


# speedup vs baseline: 1.0384x; 1.0384x over previous
import jax
import jax.numpy as jnp
from jax import lax
from jax.experimental import pallas as pl
from jax.experimental.pallas import tpu as pltpu
from jax.experimental.pallas import tpu_sc as plsc

_V = 100000
_K = 16
_L = 16
_THRESH = 10

_TCOLS = _V // 128
_TAIL = _V - _TCOLS * 128
_VPAD = (_TCOLS + 1) * 128


def _bm_select(xs, ones, neg1):
    cand = xs[0]
    cnt = ones
    for k in range(1, _K):
        xk = xs[k]
        eq = xk == cand
        dead = cnt == 0
        delta = jnp.where(eq, ones, neg1)
        cnt2 = cnt + delta
        cand = jnp.where(dead, xk, cand)
        cnt = jnp.where(dead, ones, cnt2)
    eqs = [jnp.where(xs[k] == cand, ones, neg1) for k in range(_K)]
    while len(eqs) > 1:
        eqs = [a + b for a, b in zip(eqs[::2], eqs[1::2])]
    thresh = ones * (2 * _THRESH - _K)
    return jnp.where(eqs[0] >= thresh, cand, neg1)


def _make_body(nc, nw):
    q, r = divmod(_TCOLS, nw)
    big_w, small_w = (q + 1) * 128, q * 128
    pairs = big_w // 32

    def body(in_hbm, tail_hbm, out_hbm, buf, out_v, tail_buf, tail_out):
        c = lax.axis_index("c")
        s = lax.axis_index("s")
        wid = s * nc + c
        is_big = wid < r
        col_base = jnp.where(is_big, wid * big_w,
                             r * big_w + (wid - r) * small_w)

        @pl.when(is_big)
        def _():
            pltpu.sync_copy(in_hbm.at[:, pl.ds(col_base, big_w)], buf)

        @pl.when(jnp.logical_not(is_big))
        def _():
            pltpu.sync_copy(in_hbm.at[:, pl.ds(col_base, small_w)],
                            buf.at[:, pl.ds(0, small_w)])

        ones16 = jnp.full((2 * _L,), 1, jnp.int16)
        neg16 = jnp.full((2 * _L,), -1, jnp.int16)

        @plsc.parallel_loop(0, pairs, unroll=1)
        def _pair(p):
            xs = []
            for k in range(_K):
                a = buf[k, pl.ds(p * 32, _L)]
                b = buf[k, pl.ds(p * 32 + _L, _L)]
                xs.append(plsc.pack(a, b, format=plsc.PackFormat.INTERLEAVED))
            res = _bm_select(xs, ones16, neg16)
            ra, rb = plsc.unpack(res, format=plsc.PackFormat.INTERLEAVED)
            ra = (ra << 16) >> 16
            rb = (rb << 16) >> 16
            out_v[0, pl.ds(p * 32, _L)] = ra
            out_v[0, pl.ds(p * 32 + _L, _L)] = rb

        @pl.when(is_big)
        def _():
            pltpu.sync_copy(out_v, out_hbm.at[:, pl.ds(col_base, big_w)])

        @pl.when(jnp.logical_not(is_big))
        def _():
            pltpu.sync_copy(out_v.at[:, pl.ds(0, small_w)],
                            out_hbm.at[:, pl.ds(col_base, small_w)])

        @pl.when(wid == nw - 1)
        def _():
            pltpu.sync_copy(tail_hbm, tail_buf)
            ones32 = jnp.full((_L,), 1, jnp.int32)
            neg32 = jnp.full((_L,), -1, jnp.int32)
            for g in range(_TAIL // _L):
                xs = [tail_buf[k, pl.ds(g * _L, _L)] for k in range(_K)]
                tail_out[0, pl.ds(g * _L, _L)] = _bm_select(xs, ones32, neg32)
            pltpu.sync_copy(tail_out, out_hbm.at[:, pl.ds(_TCOLS * 128, 128)])

    return body


def kernel(inputs):
    info = plsc.get_sparse_core_info()
    nc, ns = info.num_cores, info.num_subcores
    nw = nc * ns
    q, r = divmod(_TCOLS, nw)
    big_w = (q + 1) * 128

    body = _make_body(nc, nw)
    mesh = plsc.VectorSubcoreMesh(core_axis_name="c", subcore_axis_name="s")
    xt = inputs.T
    tail = xt[:, _TCOLS * 128:]
    out = pl.kernel(
        body,
        out_type=jax.ShapeDtypeStruct((1, _VPAD), jnp.int32),
        mesh=mesh,
        scratch_types=[
            pltpu.VMEM((_K, big_w), jnp.int32),
            pltpu.VMEM((1, big_w), jnp.int32),
            pltpu.VMEM((_K, _TAIL), jnp.int32),
            pltpu.VMEM((1, 128), jnp.int32),
        ],
        compiler_params=pltpu.CompilerParams(
            use_tc_tiling_on_sc=True,
            needs_layout_passes=False,
        ),
    )(xt, tail)
    return out[0, :_V].reshape(_V, 1)
